# baseline (speedup 1.0000x reference)
import functools

import jax
import jax.numpy as jnp
from jax import lax
from jax.experimental import pallas as pl
from jax.experimental.pallas import tpu as pltpu
from jax.experimental.pallas import tpu_sc as plsc

B, S, D = 4, 2048, 1024
NC, NS, L = 2, 16, 16
NW = NC * NS
S_PER_W = S // NW
R_SUB = 16
N_SUB = S_PER_W // R_SUB
CHUNK = R_SUB * D
NVEC = CHUNK // L


def _sc_body(x_hbm, t_hbm, o_hbm, xbuf, tbuf):
    wid = lax.axis_index("s") * NC + lax.axis_index("c")
    base = wid * (S_PER_W * D)
    for sub in range(N_SUB):
        off = base + sub * CHUNK
        pltpu.sync_copy(t_hbm.at[pl.ds(off, CHUNK)], tbuf)
        for b in range(B):
            boff = b * (S * D) + off
            pltpu.sync_copy(x_hbm.at[pl.ds(boff, CHUNK)], xbuf)

            def add_body(i, carry):
                plsc.addupdate(
                    xbuf.at[pl.ds(i * L, L)], tbuf[pl.ds(i * L, L)]
                )
                return carry

            lax.fori_loop(0, NVEC, add_body, 0)
            pltpu.sync_copy(xbuf, o_hbm.at[pl.ds(boff, CHUNK)])


_sc_call = functools.partial(
    pl.kernel,
    mesh=plsc.VectorSubcoreMesh(core_axis_name="c", subcore_axis_name="s"),
    out_type=jax.ShapeDtypeStruct((B * S * D,), jnp.float32),
    scratch_types=[
        pltpu.VMEM((CHUNK,), jnp.float32),
        pltpu.VMEM((CHUNK,), jnp.float32),
    ],
)


def kernel(x, pos_table):
    batch, seq_len, d_model = x.shape
    out_flat = _sc_call(_sc_body)(
        x.reshape(-1), pos_table[:seq_len].reshape(-1)
    )
    return out_flat.reshape(batch, seq_len, d_model)


# speedup vs baseline: 1.5459x; 1.5459x over previous
import functools

import jax
import jax.numpy as jnp
from jax import lax
from jax.experimental import pallas as pl
from jax.experimental.pallas import tpu as pltpu
from jax.experimental.pallas import tpu_sc as plsc

B, S, D = 4, 2048, 1024
NC, NS, L = 2, 16, 16
NW = NC * NS
S_PER_W = S // NW
R_SUB = 16
N_SUB = S_PER_W // R_SUB
CHUNK = R_SUB * D
N_ITEM = N_SUB * B


def _sc_body(x_hbm, t_hbm, o_hbm, xb0, xb1, tb0, tb1,
             si0, si1, so0, so1, st0, st1):
    wid = lax.axis_index("s") * NC + lax.axis_index("c")
    base = wid * (S_PER_W * D)
    xbufs, tbufs = (xb0, xb1), (tb0, tb1)
    sin, sout, stab = (si0, si1), (so0, so1), (st0, st1)

    def x_off(k):
        sub, b = divmod(k, B)
        return b * (S * D) + base + sub * CHUNK

    def t_off(sub):
        return base + sub * CHUNK

    in_d = [None] * N_ITEM
    out_d = [None] * N_ITEM
    t_d = [None] * N_SUB

    t_d[0] = pltpu.async_copy(
        t_hbm.at[pl.ds(t_off(0), CHUNK)], tbufs[0], stab[0])
    in_d[0] = pltpu.async_copy(
        x_hbm.at[pl.ds(x_off(0), CHUNK)], xbufs[0], sin[0])

    for k in range(N_ITEM):
        slot = k % 2
        if k + 1 < N_ITEM:
            nslot = (k + 1) % 2
            if k >= 1:
                out_d[k - 1].wait()
            sub_n, sub_k = (k + 1) // B, k // B
            if sub_n != sub_k:
                t_d[sub_n] = pltpu.async_copy(
                    t_hbm.at[pl.ds(t_off(sub_n), CHUNK)],
                    tbufs[sub_n % 2], stab[sub_n % 2])
            in_d[k + 1] = pltpu.async_copy(
                x_hbm.at[pl.ds(x_off(k + 1), CHUNK)], xbufs[nslot],
                sin[nslot])
        in_d[k].wait()
        sub, b = divmod(k, B)
        if b == 0:
            t_d[sub].wait()
        xb, tb = xbufs[slot], tbufs[sub % 2]

        @plsc.parallel_loop(0, CHUNK, step=L, unroll=8)
        def add_loop(i):
            plsc.addupdate(xb.at[pl.ds(i, L)], tb[pl.ds(i, L)])

        out_d[k] = pltpu.async_copy(
            xb, o_hbm.at[pl.ds(x_off(k), CHUNK)], sout[slot])

    out_d[N_ITEM - 2].wait()
    out_d[N_ITEM - 1].wait()


_sc_call = functools.partial(
    pl.kernel,
    mesh=plsc.VectorSubcoreMesh(core_axis_name="c", subcore_axis_name="s"),
    out_type=jax.ShapeDtypeStruct((B * S * D,), jnp.float32),
    scratch_types=[
        pltpu.VMEM((CHUNK,), jnp.float32),
        pltpu.VMEM((CHUNK,), jnp.float32),
        pltpu.VMEM((CHUNK,), jnp.float32),
        pltpu.VMEM((CHUNK,), jnp.float32),
        pltpu.SemaphoreType.DMA,
        pltpu.SemaphoreType.DMA,
        pltpu.SemaphoreType.DMA,
        pltpu.SemaphoreType.DMA,
        pltpu.SemaphoreType.DMA,
        pltpu.SemaphoreType.DMA,
    ],
)


def kernel(x, pos_table):
    batch, seq_len, d_model = x.shape
    out_flat = _sc_call(_sc_body)(
        x.reshape(-1), pos_table[:seq_len].reshape(-1)
    )
    return out_flat.reshape(batch, seq_len, d_model)


# speedup vs baseline: 3.6582x; 2.3664x over previous
import functools

import jax
import jax.numpy as jnp
from jax import lax
from jax.experimental import pallas as pl
from jax.experimental.pallas import tpu as pltpu
from jax.experimental.pallas import tpu_sc as plsc

B, S, D = 4, 2048, 1024
NC, NS, L = 2, 16, 16
NW = NC * NS
S_PER_W = S // NW
R_SUB = 16
N_SUB = S_PER_W // R_SUB
N_ITEM = N_SUB * B
VPR = D // L


def _sc_body(x_hbm, t_hbm, o_hbm, xb0, xb1, tb0, tb1,
             si0, si1, so0, so1, st0, st1):
    wid = lax.axis_index("s") * NC + lax.axis_index("c")
    row0 = wid * S_PER_W
    xbufs, tbufs = (xb0, xb1), (tb0, tb1)
    sin, sout, stab = (si0, si1), (so0, so1), (st0, st1)

    in_d = [None] * N_ITEM
    out_d = [None] * N_ITEM
    t_d = [None] * N_SUB

    def item(k):
        sub, b = divmod(k, B)
        return sub, b, row0 + sub * R_SUB

    t_d[0] = pltpu.async_copy(
        t_hbm.at[pl.ds(row0, R_SUB)], tbufs[0], stab[0])
    in_d[0] = pltpu.async_copy(
        x_hbm.at[0, pl.ds(row0, R_SUB)], xbufs[0], sin[0])

    for k in range(N_ITEM):
        slot = k % 2
        if k + 1 < N_ITEM:
            nslot = (k + 1) % 2
            if k >= 1:
                out_d[k - 1].wait()
            sub_n, b_n, r_n = item(k + 1)
            if sub_n != k // B:
                t_d[sub_n] = pltpu.async_copy(
                    t_hbm.at[pl.ds(r_n, R_SUB)],
                    tbufs[sub_n % 2], stab[sub_n % 2])
            in_d[k + 1] = pltpu.async_copy(
                x_hbm.at[b_n, pl.ds(r_n, R_SUB)], xbufs[nslot], sin[nslot])
        in_d[k].wait()
        sub, b, r = item(k)
        if b == 0:
            t_d[sub].wait()
        xb, tb = xbufs[slot], tbufs[sub % 2]

        @plsc.parallel_loop(0, R_SUB * VPR, step=1, unroll=8)
        def add_loop(i):
            row = i >> 6
            col = (i & (VPR - 1)) * L
            plsc.addupdate(xb.at[row, pl.ds(col, L)], tb[row, pl.ds(col, L)])

        out_d[k] = pltpu.async_copy(
            xb, o_hbm.at[b, pl.ds(r, R_SUB)], sout[slot])

    out_d[N_ITEM - 2].wait()
    out_d[N_ITEM - 1].wait()


_sc_call = functools.partial(
    pl.kernel,
    mesh=plsc.VectorSubcoreMesh(core_axis_name="c", subcore_axis_name="s"),
    out_type=jax.ShapeDtypeStruct((B, S, D), jnp.float32),
    scratch_types=[
        pltpu.VMEM((R_SUB, D), jnp.float32),
        pltpu.VMEM((R_SUB, D), jnp.float32),
        pltpu.VMEM((R_SUB, D), jnp.float32),
        pltpu.VMEM((R_SUB, D), jnp.float32),
        pltpu.SemaphoreType.DMA,
        pltpu.SemaphoreType.DMA,
        pltpu.SemaphoreType.DMA,
        pltpu.SemaphoreType.DMA,
        pltpu.SemaphoreType.DMA,
        pltpu.SemaphoreType.DMA,
    ],
)


def kernel(x, pos_table):
    batch, seq_len, d_model = x.shape
    return _sc_call(_sc_body)(x, pos_table[:seq_len])


# speedup vs baseline: 3.8201x; 1.0442x over previous
import functools

import jax
import jax.numpy as jnp
from jax import lax
from jax.experimental import pallas as pl
from jax.experimental.pallas import tpu as pltpu
from jax.experimental.pallas import tpu_sc as plsc

B, S, D = 4, 2048, 1024
NC, NS, L = 2, 16, 16
NW = NC * NS
S_PER_W = S // NW
R_SUB = 16
N_SUB = S_PER_W // R_SUB
N_ITEM = N_SUB * B
VPR = D // L


NBUF = 4
LEAD = 2


def _sc_body(x_hbm, t_hbm, o_hbm, xb0, xb1, xb2, xb3, tb0, tb1,
             si0, si1, si2, si3, so0, so1, so2, so3, st0, st1):
    wid = lax.axis_index("s") * NC + lax.axis_index("c")
    row0 = wid * S_PER_W
    xbufs = (xb0, xb1, xb2, xb3)
    tbufs = (tb0, tb1)
    sin = (si0, si1, si2, si3)
    sout = (so0, so1, so2, so3)
    stab = (st0, st1)

    in_d = [None] * N_ITEM
    out_d = [None] * N_ITEM
    t_d = [None] * N_SUB

    def item(k):
        sub, b = divmod(k, B)
        return sub, b, row0 + sub * R_SUB

    def issue_in(k):
        sub, b, r = item(k)
        if b == 0 and t_d[sub] is None:
            t_d[sub] = pltpu.async_copy(
                t_hbm.at[pl.ds(r, R_SUB)], tbufs[sub % 2], stab[sub % 2])
        in_d[k] = pltpu.async_copy(
            x_hbm.at[b, pl.ds(r, R_SUB)], xbufs[k % NBUF], sin[k % NBUF])

    for k in range(min(LEAD + 1, N_ITEM)):
        issue_in(k)

    for k in range(N_ITEM):
        kn = k + LEAD + 1
        if kn < N_ITEM:
            if kn - NBUF >= 0:
                out_d[kn - NBUF].wait()
            issue_in(kn)
        in_d[k].wait()
        sub, b, r = item(k)
        if b == 0:
            t_d[sub].wait()
        xb, tb = xbufs[k % NBUF], tbufs[sub % 2]

        @plsc.parallel_loop(0, R_SUB * VPR, step=1, unroll=8)
        def add_loop(i):
            row = i >> 6
            col = (i & (VPR - 1)) * L
            plsc.addupdate(xb.at[row, pl.ds(col, L)], tb[row, pl.ds(col, L)])

        out_d[k] = pltpu.async_copy(
            xb, o_hbm.at[b, pl.ds(r, R_SUB)], sout[k % NBUF])

    for k in range(max(0, N_ITEM - NBUF), N_ITEM):
        if out_d[k] is not None:
            out_d[k].wait()


_sc_call = functools.partial(
    pl.kernel,
    mesh=plsc.VectorSubcoreMesh(core_axis_name="c", subcore_axis_name="s"),
    out_type=jax.ShapeDtypeStruct((B, S, D), jnp.float32),
    scratch_types=(
        [pltpu.VMEM((R_SUB, D), jnp.float32)] * (NBUF + 2)
        + [pltpu.SemaphoreType.DMA] * (2 * NBUF + 2)
    ),
)


def kernel(x, pos_table):
    batch, seq_len, d_model = x.shape
    return _sc_call(_sc_body)(x, pos_table[:seq_len])


# speedup vs baseline: 4.1947x; 1.0981x over previous
import functools

import jax
import jax.numpy as jnp
from jax import lax
from jax.experimental import pallas as pl
from jax.experimental.pallas import tpu as pltpu
from jax.experimental.pallas import tpu_sc as plsc

B, S, D = 4, 2048, 1024
NC, NS, L = 2, 16, 16
NW = NC * NS
S_PER_W = S // NW
R_SUB = 8
NG = S_PER_W // R_SUB
VPR = D // L
NRING = 3


def _sc_body(x_hbm, t_hbm, o_hbm, *refs):
    xbufs = [[refs[r * B + b] for b in range(B)] for r in range(NRING)]
    tbufs = list(refs[NRING * B:NRING * B + NRING])
    sems = refs[NRING * B + NRING:]
    sin = [[sems[r * B + b] for b in range(B)] for r in range(NRING)]
    sout = [[sems[NRING * B + r * B + b] for b in range(B)]
            for r in range(NRING)]
    stab = list(sems[2 * NRING * B:2 * NRING * B + NRING])

    wid = lax.axis_index("s") * NC + lax.axis_index("c")
    row0 = wid * S_PER_W

    in_d = [[None] * B for _ in range(NG)]
    out_d = [[None] * B for _ in range(NG)]
    t_d = [None] * NG

    def issue_group(g):
        r = row0 + g * R_SUB
        slot = g % NRING
        t_d[g] = pltpu.async_copy(
            t_hbm.at[pl.ds(r, R_SUB)], tbufs[slot], stab[slot])
        for b in range(B):
            in_d[g][b] = pltpu.async_copy(
                x_hbm.at[b, pl.ds(r, R_SUB)], xbufs[slot][b], sin[slot][b])

    issue_group(0)
    issue_group(1)

    for g in range(NG):
        slot = g % NRING
        gn = g + 2
        if gn < NG:
            if gn - NRING >= 0:
                for b in range(B):
                    out_d[gn - NRING][b].wait()
            issue_group(gn)
        t_d[g].wait()
        for b in range(B):
            in_d[g][b].wait()
        tb = tbufs[slot]
        xa, xbb, xc, xd = xbufs[slot]

        @plsc.parallel_loop(0, R_SUB * VPR, step=1, unroll=4)
        def add_loop(i):
            row = i >> 6
            col = (i & (VPR - 1)) * L
            v = tb[row, pl.ds(col, L)]
            plsc.addupdate(xa.at[row, pl.ds(col, L)], v)
            plsc.addupdate(xbb.at[row, pl.ds(col, L)], v)
            plsc.addupdate(xc.at[row, pl.ds(col, L)], v)
            plsc.addupdate(xd.at[row, pl.ds(col, L)], v)

        r = row0 + g * R_SUB
        for b in range(B):
            out_d[g][b] = pltpu.async_copy(
                xbufs[slot][b], o_hbm.at[b, pl.ds(r, R_SUB)], sout[slot][b])

    for g in range(max(0, NG - NRING), NG):
        for b in range(B):
            if out_d[g][b] is not None:
                out_d[g][b].wait()


_sc_call = functools.partial(
    pl.kernel,
    mesh=plsc.VectorSubcoreMesh(core_axis_name="c", subcore_axis_name="s"),
    out_type=jax.ShapeDtypeStruct((B, S, D), jnp.float32),
    scratch_types=(
        [pltpu.VMEM((R_SUB, D), jnp.float32)] * (NRING * B + NRING)
        + [pltpu.SemaphoreType.DMA] * (2 * NRING * B + NRING)
    ),
)


def kernel(x, pos_table):
    batch, seq_len, d_model = x.shape
    return _sc_call(_sc_body)(x, pos_table[:seq_len])
